# baseline (speedup 1.0000x reference)
import functools

import jax
import jax.numpy as jnp
from jax import lax
from jax.experimental import pallas as pl
from jax.experimental.pallas import tpu as pltpu
from jax.experimental.pallas import tpu_sc as plsc

_NC = 2
_NS = 16
_NW = _NC * _NS

_N = 10000
_D = 128
_DH = _D // _NC
_B = 128
_N_PAD = 10240
_RPT = _N_PAD // _NS
_RPT_LAST = _N - (_NS - 1) * _RPT
_DW = 16

_NBD = 80
_NBP = 160
_RING = 4

_mesh = plsc.VectorSubcoreMesh(core_axis_name="c", subcore_axis_name="s", num_cores=_NC, num_subcores=_NS)


@functools.partial(
    pl.kernel,
    out_type=jax.ShapeDtypeStruct((_NC, _N, _DW), jnp.float32),
    mesh=_mesh,
    scratch_types=[
        pltpu.VMEM((_NBD, _B), jnp.int32),
        pltpu.VMEM((_B, _DW), jnp.float32),
        pltpu.VMEM_SHARED((_N_PAD, _DW), jnp.float32),
    ],
    compiler_params=pltpu.CompilerParams(use_tc_tiling_on_sc=False),
)
def _deg_call(colsd_hbm, ones_hbm, zerow_hbm, out_hbm, cols_v, ones_v, dacc):
  cid = lax.axis_index("c")
  sid = lax.axis_index("s")
  wid = sid * _NC + cid
  pltpu.sync_copy(colsd_hbm.at[wid], cols_v)
  pltpu.sync_copy(ones_hbm, ones_v)
  pltpu.sync_copy(zerow_hbm, dacc.at[pl.ds(sid * _RPT, _RPT)])
  plsc.subcore_barrier()

  def step(j, c):
    pltpu.sync_copy(ones_v, dacc.at[cols_v.at[j]], add=True)
    return c

  lax.fori_loop(0, _NBD, step, 0)
  plsc.subcore_barrier()
  base = sid * _RPT

  @pl.when(sid < _NS - 1)
  def _():
    pltpu.sync_copy(dacc.at[pl.ds(base, _RPT)],
                    out_hbm.at[cid, pl.ds(base, _RPT)])

  @pl.when(sid == _NS - 1)
  def _():
    pltpu.sync_copy(dacc.at[pl.ds(base, _RPT_LAST)],
                    out_hbm.at[cid, pl.ds(base, _RPT_LAST)])


@functools.partial(
    pl.kernel,
    out_type=jax.ShapeDtypeStruct((_NC, _N, _DH), jnp.float32),
    mesh=_mesh,
    scratch_types=[
        pltpu.VMEM((_NBP + _RING, _B), jnp.int32),
        pltpu.VMEM((_NBP, _B), jnp.int32),
        [pltpu.VMEM((_B, _DH), jnp.float32)] * _RING,
        pltpu.VMEM_SHARED((_N_PAD, _DH), jnp.float32),
        [pltpu.SemaphoreType.DMA] * _RING,
        [pltpu.SemaphoreType.DMA] * _RING,
    ],
    compiler_params=pltpu.CompilerParams(use_tc_tiling_on_sc=False),
)
def _prop_call(ws_hbm, rowsp_hbm, colsp_hbm, out_hbm,
               rows_v, cols_v, bufs, acc, gsems, ssems):
  cid = lax.axis_index("c")
  sid = lax.axis_index("s")
  w_half = ws_hbm.at[cid]
  pltpu.sync_copy(rowsp_hbm.at[sid], rows_v)
  pltpu.sync_copy(colsp_hbm.at[sid], cols_v)
  pltpu.sync_copy(ws_hbm.at[cid, pl.ds(sid * _RPT, _RPT)],
                  acc.at[pl.ds(sid * _RPT, _RPT)])
  plsc.subcore_barrier()

  for b in range(_RING):
    pltpu.async_copy(w_half.at[rows_v.at[b]], bufs[b], gsems[b])

  def step(i, c):
    j = i * _RING
    for b in range(_RING):
      pltpu.make_async_copy(
          w_half.at[rows_v.at[j + b]], bufs[b], gsems[b]).wait()
      pltpu.async_copy(bufs[b], acc.at[cols_v.at[j + b]], ssems[b],
                       add=True)
    for b in range(_RING):
      pltpu.make_async_copy(
          bufs[b], acc.at[cols_v.at[j + b]], ssems[b]).wait()
      pltpu.async_copy(
          w_half.at[rows_v.at[j + b + _RING]], bufs[b], gsems[b])
    return c

  lax.fori_loop(0, _NBP // _RING, step, 0)
  for b in range(_RING):
    pltpu.make_async_copy(
        w_half.at[rows_v.at[_NBP + b]], bufs[b], gsems[b]).wait()
  plsc.subcore_barrier()
  base = sid * _RPT

  @pl.when(sid < _NS - 1)
  def _():
    pltpu.sync_copy(acc.at[pl.ds(base, _RPT)],
                    out_hbm.at[cid, pl.ds(base, _RPT)])

  @pl.when(sid == _NS - 1)
  def _():
    pltpu.sync_copy(acc.at[pl.ds(base, _RPT_LAST)],
                    out_hbm.at[cid, pl.ds(base, _RPT_LAST)])


def _prep_call(dega, degb, x):
  def body(dega_ref, degb_ref, x_ref, w0_ref, s_ref, s2_ref):
    deg = dega_ref[:, 0:1] + degb_ref[:, 0:1] + 1.0
    dinv = jnp.where(deg > 0, lax.rsqrt(deg), 0.0)
    dinv2 = jnp.where(deg > 0, 1.0 / deg, 0.0)
    w0 = x_ref[...] * dinv
    w0_ref[...] = jnp.stack([w0[:, :_DH], w0[:, _DH:]])
    s_ref[...] = jnp.broadcast_to(jnp.sqrt(deg), (_N, _DH))
    s2_ref[...] = jnp.broadcast_to(dinv2, (_N, _DH))

  return pl.pallas_call(
      body,
      out_shape=(
          jax.ShapeDtypeStruct((_NC, _N, _DH), jnp.float32),
          jax.ShapeDtypeStruct((_N, _DH), jnp.float32),
          jax.ShapeDtypeStruct((_N, _DH), jnp.float32),
      ),
  )(dega, degb, x)


def _scale_call(t, s2):
  def body(t_ref, s2_ref, o_ref):
    o_ref[...] = s2_ref[...][None] * t_ref[...]

  return pl.pallas_call(
      body, out_shape=jax.ShapeDtypeStruct((_NC, _N, _DH), jnp.float32)
  )(t, s2)


def _final_call(t, s, w_mat, bias):
  def body(t_ref, s_ref, wm_ref, b_ref, o_ref):
    h = s_ref[...][None] * t_ref[...]
    wm = wm_ref[...]
    o_ref[...] = (
        lax.dot_general(h[0], wm[:, :_DH], (((1,), (1,)), ((), ())),
                        preferred_element_type=jnp.float32)
        + lax.dot_general(h[1], wm[:, _DH:], (((1,), (1,)), ((), ())),
                          preferred_element_type=jnp.float32)
        + b_ref[...]
    )

  return pl.pallas_call(
      body, out_shape=jax.ShapeDtypeStruct((_N, _D), jnp.float32)
  )(t, s, w_mat, bias)


def kernel(x, edge_index, W, b):
  row = edge_index[0].astype(jnp.int32)
  col = edge_index[1].astype(jnp.int32)
  e = row.shape[0]

  per_w = e // _NW
  padd = _NBD * _B - per_w
  colsd = jnp.pad(col.reshape(_NW, per_w), ((0, 0), (0, padd)),
                  constant_values=_N).reshape(_NW, _NBD, _B)

  per_t = e // _NS
  padp = _NBP * _B - per_t
  rowsp = jnp.pad(row.reshape(_NS, per_t), ((0, 0), (0, padp)))
  colsp = jnp.pad(col.reshape(_NS, per_t), ((0, 0), (0, padp)),
                  constant_values=_N)
  rowsp = jnp.pad(rowsp.reshape(_NS, _NBP, _B),
                  ((0, 0), (0, _RING), (0, 0)))
  colsp = colsp.reshape(_NS, _NBP, _B)

  zeros_w = jnp.zeros((_RPT, _DW), jnp.float32)
  ones_b = jnp.ones((_B, _DW), jnp.float32)
  bias = b.reshape(1, _D)

  dacc = _deg_call(colsd, ones_b, zeros_w)
  ws, s_h, s2_h = _prep_call(dacc[0], dacc[1], x)

  def round_fn(w_c, _):
    t = _prop_call(w_c, rowsp, colsp)
    return _scale_call(t, s2_h), None

  ws, _ = lax.scan(round_fn, ws, None, length=3)
  return _final_call(ws, s_h, W, bias)


# speedup vs baseline: 1.0339x; 1.0339x over previous
import functools

import jax
import jax.numpy as jnp
from jax import lax
from jax.experimental import pallas as pl
from jax.experimental.pallas import tpu as pltpu
from jax.experimental.pallas import tpu_sc as plsc

_NC = 2
_NS = 16
_NW = _NC * _NS

_N = 10000
_D = 128
_DH = _D // _NC
_B = 128
_N_PAD = 10240
_RPT = _N_PAD // _NS
_RPT_LAST = _N - (_NS - 1) * _RPT
_DW = 16

_NBD = 80
_NBP = 160
_RING = 4

_mesh = plsc.VectorSubcoreMesh(core_axis_name="c", subcore_axis_name="s", num_cores=_NC, num_subcores=_NS)


@functools.partial(
    pl.kernel,
    out_type=jax.ShapeDtypeStruct((_NC, _N, _DW), jnp.float32),
    mesh=_mesh,
    scratch_types=[
        pltpu.VMEM((_NBD, _B), jnp.int32),
        pltpu.VMEM((_B, _DW), jnp.float32),
        pltpu.VMEM_SHARED((_N_PAD, _DW), jnp.float32),
    ],
    compiler_params=pltpu.CompilerParams(use_tc_tiling_on_sc=False),
)
def _deg_call(colsd_hbm, ones_hbm, zerow_hbm, out_hbm, cols_v, ones_v, dacc):
  cid = lax.axis_index("c")
  sid = lax.axis_index("s")
  wid = sid * _NC + cid
  pltpu.sync_copy(colsd_hbm.at[wid], cols_v)
  pltpu.sync_copy(ones_hbm, ones_v)
  pltpu.sync_copy(zerow_hbm, dacc.at[pl.ds(sid * _RPT, _RPT)])
  plsc.subcore_barrier()

  def step(j, c):
    pltpu.sync_copy(ones_v, dacc.at[cols_v.at[j]], add=True)
    return c

  lax.fori_loop(0, _NBD, step, 0)
  plsc.subcore_barrier()
  base = sid * _RPT

  @pl.when(sid < _NS - 1)
  def _():
    pltpu.sync_copy(dacc.at[pl.ds(base, _RPT)],
                    out_hbm.at[cid, pl.ds(base, _RPT)])

  @pl.when(sid == _NS - 1)
  def _():
    pltpu.sync_copy(dacc.at[pl.ds(base, _RPT_LAST)],
                    out_hbm.at[cid, pl.ds(base, _RPT_LAST)])


@functools.partial(
    pl.kernel,
    out_type=jax.ShapeDtypeStruct((_NC, _N, _DH), jnp.float32),
    mesh=_mesh,
    scratch_types=[
        pltpu.VMEM((_NBP + _RING, _B), jnp.int32),
        pltpu.VMEM((_NBP, _B), jnp.int32),
        [pltpu.VMEM((_B, _DH), jnp.float32)] * _RING,
        pltpu.VMEM_SHARED((_N_PAD, _DH), jnp.float32),
        [pltpu.SemaphoreType.DMA] * _RING,
        [pltpu.SemaphoreType.DMA] * _RING,
    ],
    compiler_params=pltpu.CompilerParams(use_tc_tiling_on_sc=False),
)
def _prop_call(ws_hbm, rowsp_hbm, colsp_hbm, out_hbm,
               rows_v, cols_v, bufs, acc, gsems, ssems):
  cid = lax.axis_index("c")
  sid = lax.axis_index("s")
  w_half = ws_hbm.at[cid]
  pltpu.sync_copy(rowsp_hbm.at[sid], rows_v)
  pltpu.sync_copy(colsp_hbm.at[sid], cols_v)
  pltpu.sync_copy(ws_hbm.at[cid, pl.ds(sid * _RPT, _RPT)],
                  acc.at[pl.ds(sid * _RPT, _RPT)])
  plsc.subcore_barrier()

  for b in range(_RING):
    pltpu.async_copy(w_half.at[rows_v.at[b]], bufs[b], gsems[b])

  def step(i, c):
    j = i * _RING
    for b in range(_RING):
      pltpu.make_async_copy(
          w_half.at[rows_v.at[j + b]], bufs[b], gsems[b]).wait()
      pltpu.sync_copy(bufs[b], acc.at[cols_v.at[j + b]], add=True)
      pltpu.async_copy(
          w_half.at[rows_v.at[j + b + _RING]], bufs[b], gsems[b])
    return c

  lax.fori_loop(0, _NBP // _RING, step, 0)
  for b in range(_RING):
    pltpu.make_async_copy(
        w_half.at[rows_v.at[_NBP + b]], bufs[b], gsems[b]).wait()
  plsc.subcore_barrier()
  base = sid * _RPT

  @pl.when(sid < _NS - 1)
  def _():
    pltpu.sync_copy(acc.at[pl.ds(base, _RPT)],
                    out_hbm.at[cid, pl.ds(base, _RPT)])

  @pl.when(sid == _NS - 1)
  def _():
    pltpu.sync_copy(acc.at[pl.ds(base, _RPT_LAST)],
                    out_hbm.at[cid, pl.ds(base, _RPT_LAST)])


def _prep_call(dega, degb, x):
  def body(dega_ref, degb_ref, x_ref, w0_ref, s_ref, s2_ref):
    deg = dega_ref[:, 0:1] + degb_ref[:, 0:1] + 1.0
    dinv = jnp.where(deg > 0, lax.rsqrt(deg), 0.0)
    dinv2 = jnp.where(deg > 0, 1.0 / deg, 0.0)
    w0 = x_ref[...] * dinv
    w0_ref[...] = jnp.stack([w0[:, :_DH], w0[:, _DH:]])
    s_ref[...] = jnp.broadcast_to(jnp.sqrt(deg), (_N, _DH))
    s2_ref[...] = jnp.broadcast_to(dinv2, (_N, _DH))

  return pl.pallas_call(
      body,
      out_shape=(
          jax.ShapeDtypeStruct((_NC, _N, _DH), jnp.float32),
          jax.ShapeDtypeStruct((_N, _DH), jnp.float32),
          jax.ShapeDtypeStruct((_N, _DH), jnp.float32),
      ),
  )(dega, degb, x)


def _scale_call(t, s2):
  def body(t_ref, s2_ref, o_ref):
    o_ref[...] = s2_ref[...][None] * t_ref[...]

  return pl.pallas_call(
      body, out_shape=jax.ShapeDtypeStruct((_NC, _N, _DH), jnp.float32)
  )(t, s2)


def _final_call(t, s, w_mat, bias):
  def body(t_ref, s_ref, wm_ref, b_ref, o_ref):
    h = s_ref[...][None] * t_ref[...]
    wm = wm_ref[...]
    o_ref[...] = (
        lax.dot_general(h[0], wm[:, :_DH], (((1,), (1,)), ((), ())),
                        preferred_element_type=jnp.float32)
        + lax.dot_general(h[1], wm[:, _DH:], (((1,), (1,)), ((), ())),
                          preferred_element_type=jnp.float32)
        + b_ref[...]
    )

  return pl.pallas_call(
      body, out_shape=jax.ShapeDtypeStruct((_N, _D), jnp.float32)
  )(t, s, w_mat, bias)


def kernel(x, edge_index, W, b):
  row = edge_index[0].astype(jnp.int32)
  col = edge_index[1].astype(jnp.int32)
  e = row.shape[0]

  per_w = e // _NW
  padd = _NBD * _B - per_w
  colsd = jnp.pad(col.reshape(_NW, per_w), ((0, 0), (0, padd)),
                  constant_values=_N).reshape(_NW, _NBD, _B)

  per_t = e // _NS
  padp = _NBP * _B - per_t
  rowsp = jnp.pad(row.reshape(_NS, per_t), ((0, 0), (0, padp)))
  colsp = jnp.pad(col.reshape(_NS, per_t), ((0, 0), (0, padp)),
                  constant_values=_N)
  rowsp = jnp.pad(rowsp.reshape(_NS, _NBP, _B),
                  ((0, 0), (0, _RING), (0, 0)))
  colsp = colsp.reshape(_NS, _NBP, _B)

  zeros_w = jnp.zeros((_RPT, _DW), jnp.float32)
  ones_b = jnp.ones((_B, _DW), jnp.float32)
  bias = b.reshape(1, _D)

  dacc = _deg_call(colsd, ones_b, zeros_w)
  ws, s_h, s2_h = _prep_call(dacc[0], dacc[1], x)

  def round_fn(w_c, _):
    t = _prop_call(w_c, rowsp, colsp)
    return _scale_call(t, s2_h), None

  ws, _ = lax.scan(round_fn, ws, None, length=3)
  return _final_call(ws, s_h, W, bias)


# speedup vs baseline: 1.1592x; 1.1211x over previous
import functools

import jax
import jax.numpy as jnp
from jax import lax
from jax.experimental import pallas as pl
from jax.experimental.pallas import tpu as pltpu
from jax.experimental.pallas import tpu_sc as plsc

_NC = 2
_NS = 16
_NW = _NC * _NS

_N = 10000
_D = 128
_DH = _D // _NC
_B = 128
_N_PAD = 10240
_RPT = _N_PAD // _NS
_RPT_LAST = _N - (_NS - 1) * _RPT
_DW = 16

_NBD = 80
_NBP = 160
_RING = 2

_mesh = plsc.VectorSubcoreMesh(core_axis_name="c", subcore_axis_name="s", num_cores=_NC, num_subcores=_NS)


@functools.partial(
    pl.kernel,
    out_type=jax.ShapeDtypeStruct((_NC, _N, _DW), jnp.float32),
    mesh=_mesh,
    scratch_types=[
        pltpu.VMEM((_NBD, _B), jnp.int32),
        pltpu.VMEM((_B, _DW), jnp.float32),
        pltpu.VMEM_SHARED((_N_PAD, _DW), jnp.float32),
    ],
    compiler_params=pltpu.CompilerParams(use_tc_tiling_on_sc=False),
)
def _deg_call(colsd_hbm, ones_hbm, zerow_hbm, out_hbm, cols_v, ones_v, dacc):
  cid = lax.axis_index("c")
  sid = lax.axis_index("s")
  wid = sid * _NC + cid
  pltpu.sync_copy(colsd_hbm.at[wid], cols_v)
  pltpu.sync_copy(ones_hbm, ones_v)
  pltpu.sync_copy(zerow_hbm, dacc.at[pl.ds(sid * _RPT, _RPT)])
  plsc.subcore_barrier()

  def step(j, c):
    pltpu.sync_copy(ones_v, dacc.at[cols_v.at[j]], add=True)
    return c

  lax.fori_loop(0, _NBD, step, 0)
  plsc.subcore_barrier()
  base = sid * _RPT

  @pl.when(sid < _NS - 1)
  def _():
    pltpu.sync_copy(dacc.at[pl.ds(base, _RPT)],
                    out_hbm.at[cid, pl.ds(base, _RPT)])

  @pl.when(sid == _NS - 1)
  def _():
    pltpu.sync_copy(dacc.at[pl.ds(base, _RPT_LAST)],
                    out_hbm.at[cid, pl.ds(base, _RPT_LAST)])


@functools.partial(
    pl.kernel,
    out_type=jax.ShapeDtypeStruct((_NC, _N, _DH), jnp.float32),
    mesh=_mesh,
    scratch_types=[
        pltpu.VMEM((_NBP + _RING, _B), jnp.int32),
        pltpu.VMEM((_NBP, _B), jnp.int32),
        [pltpu.VMEM((_B, _DH), jnp.float32)] * _RING,
        pltpu.VMEM_SHARED((_N_PAD, _DH), jnp.float32),
        [pltpu.SemaphoreType.DMA] * _RING,
        [pltpu.SemaphoreType.DMA] * _RING,
    ],
    compiler_params=pltpu.CompilerParams(use_tc_tiling_on_sc=False),
)
def _prop_call(ws_hbm, rowsp_hbm, colsp_hbm, out_hbm,
               rows_v, cols_v, bufs, acc, gsems, ssems):
  cid = lax.axis_index("c")
  sid = lax.axis_index("s")
  w_half = ws_hbm.at[cid]
  pltpu.sync_copy(rowsp_hbm.at[sid], rows_v)
  pltpu.sync_copy(colsp_hbm.at[sid], cols_v)
  pltpu.sync_copy(ws_hbm.at[cid, pl.ds(sid * _RPT, _RPT)],
                  acc.at[pl.ds(sid * _RPT, _RPT)])
  plsc.subcore_barrier()

  for b in range(_RING):
    pltpu.async_copy(w_half.at[rows_v.at[b]], bufs[b], gsems[b])

  def step(i, c):
    j = i * _RING
    for b in range(_RING):
      pltpu.make_async_copy(
          w_half.at[rows_v.at[j + b]], bufs[b], gsems[b]).wait()
      pltpu.sync_copy(bufs[b], acc.at[cols_v.at[j + b]], add=True)
      pltpu.async_copy(
          w_half.at[rows_v.at[j + b + _RING]], bufs[b], gsems[b])
    return c

  lax.fori_loop(0, _NBP // _RING, step, 0)
  for b in range(_RING):
    pltpu.make_async_copy(
        w_half.at[rows_v.at[_NBP + b]], bufs[b], gsems[b]).wait()
  plsc.subcore_barrier()
  base = sid * _RPT

  @pl.when(sid < _NS - 1)
  def _():
    pltpu.sync_copy(acc.at[pl.ds(base, _RPT)],
                    out_hbm.at[cid, pl.ds(base, _RPT)])

  @pl.when(sid == _NS - 1)
  def _():
    pltpu.sync_copy(acc.at[pl.ds(base, _RPT_LAST)],
                    out_hbm.at[cid, pl.ds(base, _RPT_LAST)])


def _prep_call(dega, degb, x):
  def body(dega_ref, degb_ref, x_ref, w0_ref, s_ref, s2_ref):
    deg = dega_ref[:, 0:1] + degb_ref[:, 0:1] + 1.0
    dinv = jnp.where(deg > 0, lax.rsqrt(deg), 0.0)
    dinv2 = jnp.where(deg > 0, 1.0 / deg, 0.0)
    w0 = x_ref[...] * dinv
    w0_ref[...] = jnp.stack([w0[:, :_DH], w0[:, _DH:]])
    s_ref[...] = jnp.broadcast_to(jnp.sqrt(deg), (_N, _DH))
    s2_ref[...] = jnp.broadcast_to(dinv2, (_N, _DH))

  return pl.pallas_call(
      body,
      out_shape=(
          jax.ShapeDtypeStruct((_NC, _N, _DH), jnp.float32),
          jax.ShapeDtypeStruct((_N, _DH), jnp.float32),
          jax.ShapeDtypeStruct((_N, _DH), jnp.float32),
      ),
  )(dega, degb, x)


def _scale_call(t, s2):
  def body(t_ref, s2_ref, o_ref):
    o_ref[...] = s2_ref[...][None] * t_ref[...]

  return pl.pallas_call(
      body, out_shape=jax.ShapeDtypeStruct((_NC, _N, _DH), jnp.float32)
  )(t, s2)


def _final_call(t, s, w_mat, bias):
  def body(t_ref, s_ref, wm_ref, b_ref, o_ref):
    h = s_ref[...][None] * t_ref[...]
    wm = wm_ref[...]
    o_ref[...] = (
        lax.dot_general(h[0], wm[:, :_DH], (((1,), (1,)), ((), ())),
                        preferred_element_type=jnp.float32)
        + lax.dot_general(h[1], wm[:, _DH:], (((1,), (1,)), ((), ())),
                          preferred_element_type=jnp.float32)
        + b_ref[...]
    )

  return pl.pallas_call(
      body, out_shape=jax.ShapeDtypeStruct((_N, _D), jnp.float32)
  )(t, s, w_mat, bias)


def kernel(x, edge_index, W, b):
  row = edge_index[0].astype(jnp.int32)
  col = edge_index[1].astype(jnp.int32)
  e = row.shape[0]

  per_w = e // _NW
  padd = _NBD * _B - per_w
  colsd = jnp.pad(col.reshape(_NW, per_w), ((0, 0), (0, padd)),
                  constant_values=_N).reshape(_NW, _NBD, _B)

  per_t = e // _NS
  padp = _NBP * _B - per_t
  rowsp = jnp.pad(row.reshape(_NS, per_t), ((0, 0), (0, padp)))
  colsp = jnp.pad(col.reshape(_NS, per_t), ((0, 0), (0, padp)),
                  constant_values=_N)
  rowsp = jnp.pad(rowsp.reshape(_NS, _NBP, _B),
                  ((0, 0), (0, _RING), (0, 0)))
  colsp = colsp.reshape(_NS, _NBP, _B)

  zeros_w = jnp.zeros((_RPT, _DW), jnp.float32)
  ones_b = jnp.ones((_B, _DW), jnp.float32)
  bias = b.reshape(1, _D)

  dacc = _deg_call(colsd, ones_b, zeros_w)
  ws, s_h, s2_h = _prep_call(dacc[0], dacc[1], x)

  def round_fn(w_c, _):
    t = _prop_call(w_c, rowsp, colsp)
    return _scale_call(t, s2_h), None

  ws, _ = lax.scan(round_fn, ws, None, length=3)
  return _final_call(ws, s_h, W, bias)


# speedup vs baseline: 2.4302x; 2.0965x over previous
import functools

import jax
import jax.numpy as jnp
from jax import lax
from jax.experimental import pallas as pl
from jax.experimental.pallas import tpu as pltpu
from jax.experimental.pallas import tpu_sc as plsc

_NC = 2
_NS = 16
_NW = _NC * _NS

_N = 10000
_D = 128
_DH = _D // _NC
_B = 128
_N_PAD = 10240
_RPT = _N_PAD // _NS
_RPT_LAST = _N - (_NS - 1) * _RPT
_DW = 16
_K = 3

_NBD = 80
_NBP = 160
_NBH = _NBP // 2
_RING = 2
_NCHUNK = _RPT // _B

_mesh = plsc.VectorSubcoreMesh(core_axis_name="c", subcore_axis_name="s",
                               num_cores=_NC, num_subcores=_NS)


@functools.partial(
    pl.kernel,
    out_type=jax.ShapeDtypeStruct((_NC, _N, _DW), jnp.float32),
    mesh=_mesh,
    scratch_types=[
        pltpu.VMEM((_NBD, _B), jnp.int32),
        pltpu.VMEM((_B, _DW), jnp.float32),
        pltpu.VMEM_SHARED((_N_PAD, _DW), jnp.float32),
    ],
    compiler_params=pltpu.CompilerParams(use_tc_tiling_on_sc=False),
)
def _deg_call(colsd_hbm, ones_hbm, zerow_hbm, out_hbm, cols_v, ones_v, dacc):
  cid = lax.axis_index("c")
  sid = lax.axis_index("s")
  wid = sid * _NC + cid
  pltpu.sync_copy(colsd_hbm.at[wid], cols_v)
  pltpu.sync_copy(ones_hbm, ones_v)
  pltpu.sync_copy(zerow_hbm, dacc.at[pl.ds(sid * _RPT, _RPT)])
  plsc.subcore_barrier()

  def step(j, c):
    pltpu.sync_copy(ones_v, dacc.at[cols_v.at[j]], add=True)
    return c

  lax.fori_loop(0, _NBD, step, 0)
  plsc.subcore_barrier()
  base = sid * _RPT

  @pl.when(sid < _NS - 1)
  def _():
    pltpu.sync_copy(dacc.at[pl.ds(base, _RPT)],
                    out_hbm.at[cid, pl.ds(base, _RPT)])

  @pl.when(sid == _NS - 1)
  def _():
    pltpu.sync_copy(dacc.at[pl.ds(base, _RPT_LAST)],
                    out_hbm.at[cid, pl.ds(base, _RPT_LAST)])


@functools.partial(
    pl.kernel,
    out_type=jax.ShapeDtypeStruct((_NC, _N_PAD, _DH), jnp.float32),
    mesh=_mesh,
    scratch_types=[
        pltpu.VMEM((_NBH + _RING, _B), jnp.int32),
        pltpu.VMEM((_NBH, _B), jnp.int32),
        [pltpu.VMEM((_B, _DH), jnp.float32)] * _RING,
        pltpu.VMEM((_B, _DW), jnp.float32),
        pltpu.VMEM_SHARED((_N_PAD, _DH), jnp.float32),
        pltpu.VMEM_SHARED((_N_PAD, _DH), jnp.float32),
        [pltpu.SemaphoreType.DMA] * _RING,
    ],
    compiler_params=pltpu.CompilerParams(use_tc_tiling_on_sc=False),
)
def _prop_call(ws0_hbm, rowsp_hbm, colsp_hbm, s2_hbm, out_hbm,
               rows_v, cols_v, bufs, s2c, wsp, acc, gsems):
  cid = lax.axis_index("c")
  sid = lax.axis_index("s")
  base = sid * _RPT
  pltpu.sync_copy(ws0_hbm.at[cid, pl.ds(base, _RPT)],
                  wsp.at[pl.ds(base, _RPT)])
  pltpu.sync_copy(ws0_hbm.at[cid, pl.ds(base, _RPT)],
                  acc.at[pl.ds(base, _RPT)])
  plsc.subcore_barrier()

  for k in range(_K):
    for h in range(2):
      pltpu.sync_copy(rowsp_hbm.at[sid, pl.ds(h * _NBH, _NBH + _RING)],
                      rows_v)
      pltpu.sync_copy(colsp_hbm.at[sid, pl.ds(h * _NBH, _NBH)], cols_v)
      for b in range(_RING):
        pltpu.async_copy(wsp.at[rows_v.at[b]], bufs[b], gsems[b])

      def step(i, c):
        j = i * _RING
        for b in range(_RING):
          pltpu.make_async_copy(wsp.at[rows_v.at[j + b]], bufs[b],
                                gsems[b]).wait()
          pltpu.sync_copy(bufs[b], acc.at[cols_v.at[j + b]], add=True)
          pltpu.async_copy(wsp.at[rows_v.at[j + b + _RING]], bufs[b],
                           gsems[b])
        return c

      lax.fori_loop(0, _NBH // _RING, step, 0)
      for b in range(_RING):
        pltpu.make_async_copy(wsp.at[rows_v.at[_NBH + b]], bufs[b],
                              gsems[b]).wait()
    plsc.subcore_barrier()

    if k < _K - 1:
      for c in range(_NCHUNK):
        off = base + c * _B
        pltpu.sync_copy(acc.at[pl.ds(off, _B)], bufs[0])
        pltpu.sync_copy(s2_hbm.at[pl.ds(off, _B)], s2c)

        def scale_row(r, cc):
          s2row = s2c[r, :]
          for q in range(_DH // _DW):
            sl = pl.ds(q * _DW, _DW)
            bufs[0][r, sl] = bufs[0][r, sl] * s2row
          return cc

        lax.fori_loop(0, _B, scale_row, 0)
        pltpu.sync_copy(bufs[0], wsp.at[pl.ds(off, _B)])
        pltpu.sync_copy(bufs[0], acc.at[pl.ds(off, _B)])
      plsc.subcore_barrier()
    else:
      pltpu.sync_copy(acc.at[pl.ds(base, _RPT)],
                      out_hbm.at[cid, pl.ds(base, _RPT)])


def _prep_call(dega, degb, x):
  def body(dega_ref, degb_ref, x_ref, w0_ref, s2_ref, s_ref):
    deg = dega_ref[:, 0:1] + degb_ref[:, 0:1] + 1.0
    dinv = jnp.where(deg > 0, lax.rsqrt(deg), 0.0)
    dinv2 = jnp.where(deg > 0, 1.0 / deg, 0.0)
    w0 = x_ref[...] * dinv
    zpad = jnp.zeros((_N_PAD - _N, _DH), jnp.float32)
    w0_ref[...] = jnp.stack(
        [jnp.concatenate([w0[:, :_DH], zpad]),
         jnp.concatenate([w0[:, _DH:], zpad])])
    s2_ref[...] = jnp.concatenate(
        [jnp.broadcast_to(dinv2, (_N, _DW)),
         jnp.zeros((_N_PAD - _N, _DW), jnp.float32)])
    s_ref[...] = jnp.broadcast_to(dinv, (_N, _DH))

  return pl.pallas_call(
      body,
      out_shape=(
          jax.ShapeDtypeStruct((_NC, _N_PAD, _DH), jnp.float32),
          jax.ShapeDtypeStruct((_N_PAD, _DW), jnp.float32),
          jax.ShapeDtypeStruct((_N, _DH), jnp.float32),
      ),
  )(dega, degb, x)


def _final_call(t, s, w_mat, bias):
  def body(t_ref, s_ref, wm_ref, b_ref, o_ref):
    s_v = s_ref[...]
    wm = wm_ref[...]
    h0 = s_v * t_ref[0, :_N, :]
    h1 = s_v * t_ref[1, :_N, :]
    o_ref[...] = (
        lax.dot_general(h0, wm[:, :_DH], (((1,), (1,)), ((), ())),
                        preferred_element_type=jnp.float32)
        + lax.dot_general(h1, wm[:, _DH:], (((1,), (1,)), ((), ())),
                          preferred_element_type=jnp.float32)
        + b_ref[...]
    )

  return pl.pallas_call(
      body, out_shape=jax.ShapeDtypeStruct((_N, _D), jnp.float32)
  )(t, s, w_mat, bias)


def kernel(x, edge_index, W, b):
  row = edge_index[0].astype(jnp.int32)
  col = edge_index[1].astype(jnp.int32)
  e = row.shape[0]

  per_w = e // _NW
  padd = _NBD * _B - per_w
  colsd = jnp.pad(col.reshape(_NW, per_w), ((0, 0), (0, padd)),
                  constant_values=_N).reshape(_NW, _NBD, _B)

  per_t = e // _NS
  padp = _NBP * _B - per_t
  rowsp = jnp.pad(row.reshape(_NS, per_t), ((0, 0), (0, padp)))
  colsp = jnp.pad(col.reshape(_NS, per_t), ((0, 0), (0, padp)),
                  constant_values=_N)
  rowsp = jnp.pad(rowsp.reshape(_NS, _NBP, _B),
                  ((0, 0), (0, _RING), (0, 0)))
  colsp = colsp.reshape(_NS, _NBP, _B)

  zeros_w = jnp.zeros((_RPT, _DW), jnp.float32)
  ones_b = jnp.ones((_B, _DW), jnp.float32)
  bias = b.reshape(1, _D)

  dacc = _deg_call(colsd, ones_b, zeros_w)
  ws0, s2e, s_h = _prep_call(dacc[0], dacc[1], x)
  t = _prop_call(ws0, rowsp, colsp, s2e)
  return _final_call(t, s_h, W, bias)
